# baseline (speedup 1.0000x reference)
import functools

import jax
import jax.numpy as jnp
from jax import lax
from jax.experimental import pallas as pl
from jax.experimental.pallas import tpu as pltpu
from jax.experimental.pallas import tpu_sc as plsc

NC = 2
NS = 16
NW = NC * NS
CHUNK = 128
NBUF = 4


def _emb_lookup(idx, t128, n, d):
    rpw = n // NW
    nchunk = rpw // CHUNK
    orpw = rpw // 2
    mesh = plsc.VectorSubcoreMesh(
        core_axis_name="c", subcore_axis_name="s",
        num_cores=NC, num_subcores=NS)

    @functools.partial(
        pl.kernel,
        out_type=jax.ShapeDtypeStruct((n // 2, 2 * d), jnp.float32),
        mesh=mesh,
        scratch_types=[
            pltpu.VMEM((nchunk, CHUNK), jnp.int32),
            pltpu.VMEM((NBUF, CHUNK, 2 * d), jnp.float32),
            pltpu.SemaphoreType.DMA((NBUF,)),
            pltpu.SemaphoreType.DMA((NBUF,)),
        ],
        compiler_params=pltpu.CompilerParams(use_tc_tiling_on_sc=False),
    )
    def emb(idx_hbm, table_hbm, out_hbm, idx_v, rows_v, gsem, ssem):
        wid = lax.axis_index("s") * NC + lax.axis_index("c")
        pltpu.sync_copy(idx_hbm.at[wid], idx_v)
        base = wid * orpw

        def srcs(bf):
            return (rows_v.at[bf, pl.ds(0, CHUNK // 2), pl.ds(0, d)],
                    rows_v.at[bf, pl.ds(CHUNK // 2, CHUNK // 2), pl.ds(0, d)])

        def dsts(c):
            orow = base + c * (CHUNK // 2)
            return (out_hbm.at[pl.ds(orow, CHUNK // 2), pl.ds(0, d)],
                    out_hbm.at[pl.ds(orow, CHUNK // 2), pl.ds(d, d)])

        def round_body(r, carry):
            c0 = r * NBUF
            for bf in range(NBUF):
                @pl.when(r > 0)
                def _drain():
                    for s, t in zip(srcs(bf), dsts(0)):
                        pltpu.make_async_copy(s, t, ssem.at[bf]).wait()
                pltpu.async_copy(
                    table_hbm.at[idx_v.at[c0 + bf]],
                    rows_v.at[bf], gsem.at[bf])
            for bf in range(NBUF):
                pltpu.make_async_copy(
                    table_hbm.at[idx_v.at[c0 + bf]],
                    rows_v.at[bf], gsem.at[bf]).wait()
                for s, t in zip(srcs(bf), dsts(c0 + bf)):
                    pltpu.async_copy(s, t, ssem.at[bf])
            return carry

        lax.fori_loop(0, nchunk // NBUF, round_body, 0)
        for bf in range(NBUF):
            for s, t in zip(srcs(bf), dsts(0)):
                pltpu.make_async_copy(s, t, ssem.at[bf]).wait()

    return emb(idx, t128)


def kernel(poi_ids, table):
    b, h = poi_ids.shape
    v, d = table.shape
    n = b * h
    t128 = jnp.pad(table, ((0, 0), (0, d)))
    idx = poi_ids.astype(jnp.int32).reshape(
        NW, (n // NW) // CHUNK, CHUNK // 2, 2)
    idx = jnp.swapaxes(idx, -1, -2).reshape(NW, (n // NW) // CHUNK, CHUNK)
    out = _emb_lookup(idx, t128, n, d)
    return out.reshape(b, h, d)


# speedup vs baseline: 1.0525x; 1.0525x over previous
import functools

import jax
import jax.numpy as jnp
from jax import lax
from jax.experimental import pallas as pl
from jax.experimental.pallas import tpu as pltpu
from jax.experimental.pallas import tpu_sc as plsc

NC = 2
NS = 16
NW = NC * NS
CHUNK = 128
NBUF = 4


def _emb_lookup(idx, t128, n, d):
    rpw = n // NW
    nchunk = rpw // CHUNK
    mesh = plsc.VectorSubcoreMesh(
        core_axis_name="c", subcore_axis_name="s",
        num_cores=NC, num_subcores=NS)

    @functools.partial(
        pl.kernel,
        out_type=jax.ShapeDtypeStruct((n, 2 * d), jnp.float32),
        mesh=mesh,
        scratch_types=[
            pltpu.VMEM((nchunk, CHUNK), jnp.int32),
            pltpu.VMEM((NBUF, CHUNK, 2 * d), jnp.float32),
            pltpu.SemaphoreType.DMA((NBUF,)),
            pltpu.SemaphoreType.DMA((NBUF,)),
        ],
        compiler_params=pltpu.CompilerParams(use_tc_tiling_on_sc=False),
    )
    def emb(idx_hbm, table_hbm, out_hbm, idx_v, rows_v, gsem, ssem):
        wid = lax.axis_index("s") * NC + lax.axis_index("c")
        pltpu.sync_copy(idx_hbm.at[wid], idx_v)
        base = wid * rpw

        def round_body(r, carry):
            c0 = r * NBUF
            for bf in range(NBUF):
                @pl.when(r > 0)
                def _drain():
                    pltpu.make_async_copy(
                        rows_v.at[bf],
                        out_hbm.at[pl.ds(base, CHUNK)],
                        ssem.at[bf]).wait()
                pltpu.async_copy(
                    table_hbm.at[idx_v.at[c0 + bf]],
                    rows_v.at[bf], gsem.at[bf])
            for bf in range(NBUF):
                pltpu.make_async_copy(
                    table_hbm.at[idx_v.at[c0 + bf]],
                    rows_v.at[bf], gsem.at[bf]).wait()
                pltpu.async_copy(
                    rows_v.at[bf],
                    out_hbm.at[pl.ds(base + (c0 + bf) * CHUNK, CHUNK)],
                    ssem.at[bf])
            return carry

        lax.fori_loop(0, nchunk // NBUF, round_body, 0)
        for bf in range(NBUF):
            pltpu.make_async_copy(
                rows_v.at[bf],
                out_hbm.at[pl.ds(base, CHUNK)],
                ssem.at[bf]).wait()

    return emb(idx, t128)


def kernel(poi_ids, table):
    b, h = poi_ids.shape
    v, d = table.shape
    n = b * h
    one = (poi_ids[0, 0] * 0 + 1).astype(jnp.float32)
    t128 = jnp.pad(table, ((0, 0), (0, d))) * one
    idx = poi_ids.astype(jnp.int32).reshape(NW, (n // NW) // CHUNK, CHUNK)
    out = _emb_lookup(idx, t128, n, d)
    return (out[:, :d] * one).reshape(b, h, d)


# speedup vs baseline: 1.1968x; 1.1371x over previous
import functools

import jax
import jax.numpy as jnp
from jax import lax
from jax.experimental import pallas as pl
from jax.experimental.pallas import tpu as pltpu
from jax.experimental.pallas import tpu_sc as plsc

NC = 2
NS = 16
NW = NC * NS
CHUNK = 128


def _emb_lookup(idx, table, n, d):
    rpw = n // NW
    nchunk = rpw // CHUNK
    mesh = plsc.VectorSubcoreMesh(
        core_axis_name="c", subcore_axis_name="s",
        num_cores=NC, num_subcores=NS)

    nbuf = 10
    nround = nchunk // nbuf

    @functools.partial(
        pl.kernel,
        out_type=jax.ShapeDtypeStruct((n, d), jnp.float32),
        mesh=mesh,
        scratch_types=[
            pltpu.VMEM((nchunk, CHUNK), jnp.int32),
            pltpu.VMEM((nbuf, CHUNK, d), jnp.float32),
            pltpu.SemaphoreType.DMA((nbuf,)),
            pltpu.SemaphoreType.DMA((nbuf,)),
        ],
        compiler_params=pltpu.CompilerParams(use_tc_tiling_on_sc=False),
    )
    def emb(idx_hbm, table_hbm, out_hbm, idx_v, rows_v, gsem, ssem):
        wid = lax.axis_index("s") * NC + lax.axis_index("c")
        pltpu.sync_copy(idx_hbm.at[wid], idx_v)
        base = wid * rpw

        def round_body(r, carry):
            c0 = r * nbuf
            for b in range(nbuf):
                @pl.when(r > 0)
                def _drain():
                    pltpu.make_async_copy(
                        rows_v.at[b],
                        out_hbm.at[pl.ds(base, CHUNK)],
                        ssem.at[b],
                    ).wait()
                pltpu.async_copy(
                    table_hbm.at[idx_v.at[c0 + b]], rows_v.at[b], gsem.at[b])
            for b in range(nbuf):
                pltpu.make_async_copy(
                    table_hbm.at[idx_v.at[c0 + b]], rows_v.at[b], gsem.at[b]
                ).wait()
                pltpu.async_copy(
                    rows_v.at[b],
                    out_hbm.at[pl.ds(base + (c0 + b) * CHUNK, CHUNK)],
                    ssem.at[b])
            return carry

        lax.fori_loop(0, nround, round_body, 0)
        for b in range(nbuf):
            pltpu.make_async_copy(
                rows_v.at[b],
                out_hbm.at[pl.ds(base, CHUNK)],
                ssem.at[b],
            ).wait()

    return emb(idx, table)


def kernel(poi_ids, table):
    b, h = poi_ids.shape
    v, d = table.shape
    n = b * h
    idx = poi_ids.reshape(NW, n // (NW * CHUNK), CHUNK).astype(jnp.int32)
    out = _emb_lookup(idx, table, n, d)
    return out.reshape(b, h, d)
